# Initial kernel scaffold; baseline (speedup 1.0000x reference)
import jax
import jax.numpy as jnp
from jax.experimental import pallas as pl


def kernel(shape_id, colour_id, pos_id, edge_index, edge_type, batch, shape_emb, col_emb, pos_emb, W_rel1, W_root1, b1, bn1_g, bn1_b, W_rel2, W_root2, b2, bn2_g, bn2_b, lin_W, lin_b):
    raise NotImplementedError("write your pallas kernel here")


# speedup vs baseline: 2.3102x; 2.3102x over previous
import functools

import jax
import jax.numpy as jnp
from jax import lax
from jax.experimental import pallas as pl
from jax.experimental.pallas import tpu as pltpu
from jax.experimental.pallas import tpu_sc as plsc

N = 50000
E = 800000
DIM = 64
HID = 96
NREL = 3
NGRAPH = 1024

NC = 2
NS = 16
NW = NC * NS

EPW = E // NW
EB = 1000
NBLK = EPW // EB

RAGG = 150016
RT = RAGG // NS
ZR = 1172

NPAD = 50176
NPW = NPAD // NW
NB = 784

RPOOL = 1056
PT = RPOOL // NS


def _sc_mesh():
  return plsc.VectorSubcoreMesh(core_axis_name="c", subcore_axis_name="s")


_SC_PARAMS = pltpu.CompilerParams(use_tc_tiling_on_sc=False)


@functools.partial(
    pl.kernel,
    out_type=jax.ShapeDtypeStruct((NPAD, DIM), jnp.float32),
    mesh=_sc_mesh(),
    compiler_params=_SC_PARAMS,
    scratch_types=[
        pltpu.VMEM((NB,), jnp.int32),
        pltpu.VMEM((NB, DIM), jnp.float32),
    ],
)
def _sc_embed(sid_h, cid_h, pid_h, semb_h, cemb_h, pemb_h, x_h,
              idxv, rows0):
  ci = lax.axis_index("c")
  si = lax.axis_index("s")
  w = ci * NS + si

  def body(j, carry):
    base = w * NPW + j * NB
    pltpu.sync_copy(sid_h.at[pl.ds(base, NB)], idxv)
    pltpu.sync_copy(semb_h.at[idxv], rows0)
    pltpu.sync_copy(cid_h.at[pl.ds(base, NB)], idxv)
    pltpu.sync_copy(cemb_h.at[idxv], rows0, add=True)
    pltpu.sync_copy(pid_h.at[pl.ds(base, NB)], idxv)
    pltpu.sync_copy(pemb_h.at[idxv], rows0, add=True)
    pltpu.sync_copy(rows0, x_h.at[pl.ds(base, NB)])
    return carry

  lax.fori_loop(0, NPW // NB, body, 0)


@functools.partial(
    pl.kernel,
    out_type=jax.ShapeDtypeStruct((NC, RAGG, 8), jnp.float32),
    mesh=_sc_mesh(),
    compiler_params=_SC_PARAMS,
    scratch_types=[
        pltpu.VMEM((EB,), jnp.int32),
        pltpu.VMEM((EB, 8), jnp.float32),
        pltpu.VMEM((ZR, 8), jnp.float32),
        pltpu.VMEM_SHARED((RAGG, 8), jnp.float32),
    ],
)
def _sc_counts(aidx_h, ones_h, zer_h, out_h, aidxv, onesv, zbuf, acc):
  ci = lax.axis_index("c")
  si = lax.axis_index("s")
  w = ci * NS + si
  r0 = si * RT

  pltpu.sync_copy(ones_h, onesv)
  pltpu.sync_copy(zer_h, zbuf)
  for z in range(RT // ZR):
    pltpu.sync_copy(zbuf, acc.at[pl.ds(r0 + z * ZR, ZR)])
  plsc.subcore_barrier()

  def body(j, carry):
    base = w * EPW + j * EB
    pltpu.sync_copy(aidx_h.at[pl.ds(base, EB)], aidxv)
    pltpu.sync_copy(onesv, acc.at[aidxv], add=True)
    return carry

  lax.fori_loop(0, NBLK, body, 0)
  plsc.subcore_barrier()
  for z in range(RT // ZR):
    pltpu.sync_copy(acc.at[pl.ds(r0 + z * ZR, ZR)],
                    out_h.at[ci, pl.ds(r0 + z * ZR, ZR)])


def _make_edge_agg(nch, dc):
  @functools.partial(
      pl.kernel,
      out_type=jax.ShapeDtypeStruct((nch, NC, RAGG, dc), jnp.float32),
      mesh=_sc_mesh(),
      compiler_params=_SC_PARAMS,
      scratch_types=[
          pltpu.VMEM((EB,), jnp.int32),
          pltpu.VMEM((EB,), jnp.int32),
          pltpu.VMEM((EB, dc), jnp.float32),
          pltpu.VMEM((ZR, dc), jnp.float32),
          pltpu.VMEM_SHARED((RAGG, dc), jnp.float32),
      ],
  )
  def _sc_edge_agg(xt_h, src_h, aidx_h, zer_h, out_h,
                   srcv, aidxv, rows, zbuf, acc):
    ci = lax.axis_index("c")
    si = lax.axis_index("s")
    w = ci * NS + si
    r0 = si * RT
    pltpu.sync_copy(zer_h, zbuf)
    for ch in range(nch):
      for z in range(RT // ZR):
        pltpu.sync_copy(zbuf, acc.at[pl.ds(r0 + z * ZR, ZR)])
      plsc.subcore_barrier()

      def body(j, carry):
        base = w * EPW + j * EB
        pltpu.sync_copy(src_h.at[pl.ds(base, EB)], srcv)
        pltpu.sync_copy(aidx_h.at[pl.ds(base, EB)], aidxv)
        pltpu.sync_copy(xt_h.at[ch].at[srcv], rows)
        pltpu.sync_copy(rows, acc.at[aidxv], add=True)
        return carry

      lax.fori_loop(0, NBLK, body, 0)
      plsc.subcore_barrier()
      for z in range(RT // ZR):
        pltpu.sync_copy(acc.at[pl.ds(r0 + z * ZR, ZR)],
                        out_h.at[ch, ci, pl.ds(r0 + z * ZR, ZR)])
      plsc.subcore_barrier()

  return _sc_edge_agg


_edge_agg_l1 = _make_edge_agg(8, 8)
_edge_agg_l2 = _make_edge_agg(12, 8)


@functools.partial(
    pl.kernel,
    out_type=[
        jax.ShapeDtypeStruct((NC, RPOOL, HID), jnp.float32),
        jax.ShapeDtypeStruct((NC, RPOOL, 8), jnp.float32),
    ],
    mesh=_sc_mesh(),
    compiler_params=_SC_PARAMS,
    scratch_types=[
        pltpu.VMEM((NB,), jnp.int32),
        pltpu.VMEM((NB, HID), jnp.float32),
        pltpu.VMEM((NB, 8), jnp.float32),
        pltpu.VMEM((PT, HID), jnp.float32),
        pltpu.VMEM((PT, 8), jnp.float32),
        pltpu.VMEM_SHARED((RPOOL, HID), jnp.float32),
        pltpu.VMEM_SHARED((RPOOL, 8), jnp.float32),
    ],
)
def _sc_pool(h_h, bat_h, ones_h, zs_h, zc_h, outs_h, outc_h,
             bidxv, hbuf, onesv, zsv, zcv, accs, accc):
  ci = lax.axis_index("c")
  si = lax.axis_index("s")
  w = ci * NS + si
  r0 = si * PT

  pltpu.sync_copy(ones_h, onesv)
  pltpu.sync_copy(zs_h, zsv)
  pltpu.sync_copy(zc_h, zcv)
  pltpu.sync_copy(zsv, accs.at[pl.ds(r0, PT)])
  pltpu.sync_copy(zcv, accc.at[pl.ds(r0, PT)])
  plsc.subcore_barrier()

  def body(j, carry):
    base = w * NPW + j * NB
    pltpu.sync_copy(bat_h.at[pl.ds(base, NB)], bidxv)
    pltpu.sync_copy(h_h.at[pl.ds(base, NB)], hbuf)
    pltpu.sync_copy(hbuf, accs.at[bidxv], add=True)
    pltpu.sync_copy(onesv, accc.at[bidxv], add=True)
    return carry

  lax.fori_loop(0, NPW // NB, body, 0)
  plsc.subcore_barrier()
  pltpu.sync_copy(accs.at[pl.ds(r0, PT)], outs_h.at[ci, pl.ds(r0, PT)])
  pltpu.sync_copy(accc.at[pl.ds(r0, PT)], outc_h.at[ci, pl.ds(r0, PT)])


def _tc_eidx(et2, dst2):
  def body(et_r, dst_r, out_r):
    out_r[...] = et_r[...] * N + dst_r[...]

  return pl.pallas_call(
      body,
      out_shape=jax.ShapeDtypeStruct(et2.shape, jnp.int32),
  )(et2, dst2)


def _tc_layer(x, agg, cnt, w_root, w_rel, b):
  n, d = x.shape
  blk = 400
  nsteps = n // blk

  def body(x_r, agg_r, cnt_r, wro_r, wre_r, b_r, h_r, st_r):
    i = pl.program_id(0)
    h = jnp.dot(x_r[...], wro_r[...], preferred_element_type=jnp.float32)
    h = h + b_r[...]
    icnt = 1.0 / jnp.maximum(cnt_r[...], 1.0)
    for r in range(NREL):
      a = agg_r[r] * icnt[:, r][:, None]
      h = h + jnp.dot(a, wre_r[r], preferred_element_type=jnp.float32)
    h_r[...] = h
    s = jnp.sum(h, axis=0, keepdims=True)
    ss = jnp.sum(h * h, axis=0, keepdims=True)
    st = jnp.concatenate([s, ss, jnp.zeros((6, HID), jnp.float32)], axis=0)

    @pl.when(i == 0)
    def _():
      st_r[...] = jnp.zeros_like(st_r)

    st_r[...] += st

  return pl.pallas_call(
      body,
      grid=(nsteps,),
      in_specs=[
          pl.BlockSpec((blk, d), lambda i: (i, 0)),
          pl.BlockSpec((NREL, blk, d), lambda i: (0, i, 0)),
          pl.BlockSpec((blk, 8), lambda i: (i, 0)),
          pl.BlockSpec((d, HID), lambda i: (0, 0)),
          pl.BlockSpec((NREL, d, HID), lambda i: (0, 0, 0)),
          pl.BlockSpec((1, HID), lambda i: (0, 0)),
      ],
      out_specs=[
          pl.BlockSpec((blk, HID), lambda i: (i, 0)),
          pl.BlockSpec((8, HID), lambda i: (0, 0)),
      ],
      out_shape=[
          jax.ShapeDtypeStruct((n, HID), jnp.float32),
          jax.ShapeDtypeStruct((8, HID), jnp.float32),
      ],
  )(x, agg, cnt, w_root, w_rel, b)


def _tc_norm_relu(h_pre, scale, shift):
  n = h_pre.shape[0]
  blk = 400

  def body(h_r, sc_r, sh_r, out_r):
    out_r[...] = jnp.maximum(h_r[...] * sc_r[...] + sh_r[...], 0.0)

  return pl.pallas_call(
      body,
      grid=(n // blk,),
      in_specs=[
          pl.BlockSpec((blk, HID), lambda i: (i, 0)),
          pl.BlockSpec((1, HID), lambda i: (0, 0)),
          pl.BlockSpec((1, HID), lambda i: (0, 0)),
      ],
      out_specs=pl.BlockSpec((blk, HID), lambda i: (i, 0)),
      out_shape=jax.ShapeDtypeStruct((n, HID), jnp.float32),
  )(h_pre, scale, shift)


def _tc_final(sums, cnt8, lin_w_pad, lin_b_pad):
  def body(s_r, c_r, w_r, b_r, out_r):
    icnt = 1.0 / jnp.maximum(c_r[..., :1], 1.0)
    pooled = s_r[...] * icnt
    out_r[...] = jnp.dot(pooled, w_r[...],
                         preferred_element_type=jnp.float32) + b_r[...]

  return pl.pallas_call(
      body,
      in_specs=[
          pl.BlockSpec((NGRAPH, HID), lambda: (0, 0)),
          pl.BlockSpec((NGRAPH, 8), lambda: (0, 0)),
          pl.BlockSpec((HID, 128), lambda: (0, 0)),
          pl.BlockSpec((1, 128), lambda: (0, 0)),
      ],
      out_specs=pl.BlockSpec((NGRAPH, 128), lambda: (0, 0)),
      out_shape=jax.ShapeDtypeStruct((NGRAPH, 128), jnp.float32),
  )(sums, cnt8, lin_w_pad, lin_b_pad)


def _chunked(x, nch, dc):
  n = x.shape[0]
  return x.reshape(n, nch, dc).transpose(1, 0, 2)


def _bn_scale_shift(stats, n, gamma, beta, eps=1e-5):
  s, ss = stats[0], stats[1]
  mu = s / n
  var = jnp.maximum(ss / n - mu * mu, 0.0)
  scale = gamma / jnp.sqrt(var + eps)
  shift = beta - mu * scale
  return scale.reshape(1, HID), shift.reshape(1, HID)


def kernel(shape_id, colour_id, pos_id, edge_index, edge_type, batch,
           shape_emb, col_emb, pos_emb,
           W_rel1, W_root1, b1, bn1_g, bn1_b,
           W_rel2, W_root2, b2, bn2_g, bn2_b,
           lin_W, lin_b):
  f32 = jnp.float32
  src = edge_index[0].astype(jnp.int32)
  dst = edge_index[1].astype(jnp.int32)
  et = edge_type.astype(jnp.int32)

  aidx = _tc_eidx(et.reshape(800, 1000), dst.reshape(800, 1000)).reshape(E)

  pad_n = NPAD - N
  sid_p = jnp.pad(shape_id.astype(jnp.int32), (0, pad_n))
  cid_p = jnp.pad(colour_id.astype(jnp.int32), (0, pad_n))
  pid_p = jnp.pad(jnp.clip(pos_id, 0, pos_emb.shape[0] - 1).astype(jnp.int32),
                  (0, pad_n))
  x = _sc_embed(sid_p, cid_p, pid_p,
                shape_emb.astype(f32), col_emb.astype(f32),
                pos_emb.astype(f32))[:N]

  ones_eb = jnp.ones((EB, 8), f32)
  zer_cnt = jnp.zeros((ZR, 8), f32)
  cnt8 = _sc_counts(aidx, ones_eb, zer_cnt)
  cnt = (cnt8[0] + cnt8[1])[:NREL * N, 0].reshape(NREL, N)
  cnt_n8 = jnp.pad(cnt.T, ((0, 0), (0, 8 - NREL)))

  xt = _chunked(x, 8, 8)
  aggp = _edge_agg_l1(xt, src, aidx, jnp.zeros((ZR, 8), f32))
  agg = (aggp[:, 0] + aggp[:, 1])[:, :NREL * N, :]
  agg = agg.reshape(8, NREL, N, 8).transpose(1, 2, 0, 3).reshape(NREL, N, DIM)
  h_pre, st1 = _tc_layer(x, agg, cnt_n8, W_root1.astype(f32),
                         W_rel1.astype(f32), b1.astype(f32).reshape(1, HID))
  sc1, sh1 = _bn_scale_shift(st1, N, bn1_g.astype(f32), bn1_b.astype(f32))
  h1 = _tc_norm_relu(h_pre, sc1, sh1)

  ht = _chunked(h1, 12, 8)
  aggp2 = _edge_agg_l2(ht, src, aidx, jnp.zeros((ZR, 8), f32))
  agg2 = (aggp2[:, 0] + aggp2[:, 1])[:, :NREL * N, :]
  agg2 = agg2.reshape(12, NREL, N, 8).transpose(1, 2, 0, 3).reshape(
      NREL, N, HID)
  h2_pre, st2 = _tc_layer(h1, agg2, cnt_n8, W_root2.astype(f32),
                          W_rel2.astype(f32), b2.astype(f32).reshape(1, HID))
  sc2, sh2 = _bn_scale_shift(st2, N, bn2_g.astype(f32), bn2_b.astype(f32))
  h2 = _tc_norm_relu(h2_pre, sc2, sh2)

  h2_p = jnp.pad(h2, ((0, pad_n), (0, 0)))
  bat_p = jnp.pad(batch.astype(jnp.int32), (0, pad_n),
                  constant_values=NGRAPH)
  psums, pcnt = _sc_pool(h2_p, bat_p, jnp.ones((NB, 8), f32),
                         jnp.zeros((PT, HID), f32), jnp.zeros((PT, 8), f32))
  sums = (psums[0] + psums[1])[:NGRAPH]
  cntg = (pcnt[0] + pcnt[1])[:NGRAPH]

  lin_w_pad = jnp.pad(lin_W.astype(f32), ((0, 0), (0, 128 - lin_W.shape[1])))
  lin_b_pad = jnp.pad(lin_b.astype(f32), (0, 128 - lin_b.shape[0])).reshape(
      1, 128)
  out = _tc_final(sums, cntg, lin_w_pad, lin_b_pad)
  return out[:, :lin_W.shape[1]]
